# baseline (speedup 1.0000x reference)
import jax
import jax.numpy as jnp
from jax import lax
from jax.experimental import pallas as pl
from jax.experimental.pallas import tpu as pltpu
from jax.experimental.pallas import tpu_sc as plsc

_NC = 2
_NS = 16
_LANES = 16
_NW = _NC * _NS


def _quantize_body(x_hbm, levels_hbm, q_hbm, idx_hbm, x_v, q_v, idx_v):
    chunk = x_v.shape[0]
    wid = lax.axis_index("s") * _NC + lax.axis_index("c")
    base = wid * chunk
    pltpu.sync_copy(x_hbm.at[pl.ds(base, chunk)], x_v)

    @plsc.parallel_loop(0, chunk, _LANES, unroll=16)
    def body(off):
        xv = x_v[pl.ds(off, _LANES)]
        u = 255.0 / (1.0 + jnp.exp(xv * -2.0))
        iv = (u + 0.5).astype(jnp.int32)
        q_v[pl.ds(off, _LANES)] = iv.astype(jnp.float32) * (1.0 / 127.5) - 1.0
        idx_v[pl.ds(off, _LANES)] = iv

    pltpu.sync_copy(q_v, q_hbm.at[pl.ds(base, chunk)])
    pltpu.sync_copy(idx_v, idx_hbm.at[pl.ds(base, chunk)])


def kernel(x, levels):
    n = x.shape[0]
    chunk = n // _NW
    xf = x.reshape(n)
    q, idx = pl.kernel(
        _quantize_body,
        out_type=[
            jax.ShapeDtypeStruct((n,), jnp.float32),
            jax.ShapeDtypeStruct((n,), jnp.int32),
        ],
        mesh=plsc.VectorSubcoreMesh(
            core_axis_name="c", subcore_axis_name="s",
            num_cores=_NC, num_subcores=_NS,
        ),
        scratch_types=[
            pltpu.VMEM((chunk,), jnp.float32),
            pltpu.VMEM((chunk,), jnp.float32),
            pltpu.VMEM((chunk,), jnp.int32),
        ],
    )(xf, levels)
    return q.reshape(n, 1), idx.reshape(n, 1)


# speedup vs baseline: 1.0079x; 1.0079x over previous
import jax
import jax.numpy as jnp
from jax import lax
from jax.experimental import pallas as pl
from jax.experimental.pallas import tpu as pltpu
from jax.experimental.pallas import tpu_sc as plsc

_NC = 2
_NS = 16
_LANES = 16
_NW = _NC * _NS


def _quantize_body(x_hbm, levels_hbm, q_hbm, idx_hbm, x_v, q_v, idx_v, sem):
    chunk = x_v.shape[0]
    wid = lax.axis_index("s") * _NC + lax.axis_index("c")
    base = wid * chunk
    pltpu.sync_copy(x_hbm.at[pl.ds(base, chunk)], x_v)

    @plsc.parallel_loop(0, chunk, _LANES, unroll=8)
    def body(off):
        xv = x_v[pl.ds(off, _LANES)]
        u = 255.0 / (1.0 + jnp.exp(xv * -2.0))
        iv = (u + 0.5).astype(jnp.int32)
        q_v[pl.ds(off, _LANES)] = iv.astype(jnp.float32) * (1.0 / 127.5) - 1.0
        idx_v[pl.ds(off, _LANES)] = iv

    c_q = pltpu.async_copy(q_v, q_hbm.at[pl.ds(base, chunk)], sem)
    c_i = pltpu.async_copy(idx_v, idx_hbm.at[pl.ds(base, chunk)], sem)
    c_q.wait()
    c_i.wait()


def kernel(x, levels):
    n = x.shape[0]
    chunk = n // _NW
    xf = x.reshape(n)
    q, idx = pl.kernel(
        _quantize_body,
        out_type=[
            jax.ShapeDtypeStruct((n,), jnp.float32),
            jax.ShapeDtypeStruct((n,), jnp.int32),
        ],
        mesh=plsc.VectorSubcoreMesh(
            core_axis_name="c", subcore_axis_name="s",
            num_cores=_NC, num_subcores=_NS,
        ),
        scratch_types=[
            pltpu.VMEM((chunk,), jnp.float32),
            pltpu.VMEM((chunk,), jnp.float32),
            pltpu.VMEM((chunk,), jnp.int32),
            pltpu.SemaphoreType.DMA,
        ],
    )(xf, levels)
    return q.reshape(n, 1), idx.reshape(n, 1)


# speedup vs baseline: 1.0083x; 1.0004x over previous
import jax
import jax.numpy as jnp
from jax import lax
from jax.experimental import pallas as pl
from jax.experimental.pallas import tpu as pltpu
from jax.experimental.pallas import tpu_sc as plsc

_NC = 2
_NS = 16
_LANES = 16
_NW = _NC * _NS


def _quantize_body(x_hbm, levels_hbm, q_hbm, idx_hbm, x_v, q_v, idx_v, sem):
    chunk = x_v.shape[0]
    wid = lax.axis_index("s") * _NC + lax.axis_index("c")
    base = wid * chunk
    pltpu.sync_copy(x_hbm.at[pl.ds(base, chunk)], x_v)
    half = chunk // 2

    copies = []
    for h in range(2):
        lo = h * half

        @plsc.parallel_loop(lo, lo + half, _LANES, unroll=8)
        def body(off):
            xv = x_v[pl.ds(off, _LANES)]
            u = 255.0 / (1.0 + jnp.exp(xv * -2.0))
            iv = (u + 0.5).astype(jnp.int32)
            q_v[pl.ds(off, _LANES)] = iv.astype(jnp.float32) * (1.0 / 127.5) - 1.0
            idx_v[pl.ds(off, _LANES)] = iv

        copies.append(pltpu.async_copy(
            q_v.at[pl.ds(lo, half)], q_hbm.at[pl.ds(base + lo, half)], sem))
        copies.append(pltpu.async_copy(
            idx_v.at[pl.ds(lo, half)], idx_hbm.at[pl.ds(base + lo, half)], sem))
    for c in copies:
        c.wait()


def kernel(x, levels):
    n = x.shape[0]
    chunk = n // _NW
    xf = x.reshape(n)
    q, idx = pl.kernel(
        _quantize_body,
        out_type=[
            jax.ShapeDtypeStruct((n,), jnp.float32),
            jax.ShapeDtypeStruct((n,), jnp.int32),
        ],
        mesh=plsc.VectorSubcoreMesh(
            core_axis_name="c", subcore_axis_name="s",
            num_cores=_NC, num_subcores=_NS,
        ),
        scratch_types=[
            pltpu.VMEM((chunk,), jnp.float32),
            pltpu.VMEM((chunk,), jnp.float32),
            pltpu.VMEM((chunk,), jnp.int32),
            pltpu.SemaphoreType.DMA,
        ],
    )(xf, levels)
    return q.reshape(n, 1), idx.reshape(n, 1)
